# baseline (speedup 1.0000x reference)
import functools

import jax
import jax.numpy as jnp
from jax import lax
from jax.experimental import pallas as pl
from jax.experimental.pallas import tpu as pltpu
from jax.experimental.pallas import tpu_sc as plsc

_NUM_CORES = 2
_NUM_SUBCORES = 16
_NW = _NUM_CORES * _NUM_SUBCORES

_B = 16384
_D = 32
_CHUNK = 128
_BPW = _B // _NW
_CPW = _BPW // _CHUNK


@functools.partial(
    pl.kernel,
    mesh=plsc.VectorSubcoreMesh(core_axis_name="c", subcore_axis_name="s"),
    out_type=jax.ShapeDtypeStruct((_B, _D), jnp.float32),
    scratch_types=[
        pltpu.VMEM((_CPW, _CHUNK), jnp.int32),
        pltpu.VMEM((_BPW, _D), jnp.float32),
        pltpu.SemaphoreType.DMA,
    ],
    compiler_params=pltpu.CompilerParams(use_tc_tiling_on_sc=False),
)
def _gather_rows(table_hbm, idx_hbm, out_hbm, idx_v, rows_v, sem):
    wid = lax.axis_index("s") * _NUM_CORES + lax.axis_index("c")
    pltpu.sync_copy(idx_hbm.at[wid], idx_v)
    copies = [
        pltpu.async_copy(
            table_hbm.at[idx_v.at[j]],
            rows_v.at[pl.ds(j * _CHUNK, _CHUNK)],
            sem,
        )
        for j in range(_CPW)
    ]
    for c in copies:
        c.wait()
    pltpu.sync_copy(rows_v, out_hbm.at[pl.ds(wid * _BPW, _BPW)])


def kernel(indices, values):
    idx = indices.astype(jnp.int32).reshape(_NW, _CPW, _CHUNK)
    return _gather_rows(values, idx)


# speedup vs baseline: 4.6561x; 4.6561x over previous
import functools

import jax
import jax.numpy as jnp
from jax import lax
from jax.experimental import pallas as pl
from jax.experimental.pallas import tpu as pltpu
from jax.experimental.pallas import tpu_sc as plsc

_NUM_CORES = 2
_NUM_SUBCORES = 16
_NW = _NUM_CORES * _NUM_SUBCORES

_B = 16384
_D = 32
_BPW = _B // _NW
_L = 16
_TL = 128
_R = 16


@functools.partial(
    pl.kernel,
    mesh=plsc.VectorSubcoreMesh(core_axis_name="c", subcore_axis_name="s"),
    out_type=jax.ShapeDtypeStruct((_D, _B), jnp.float32),
    scratch_types=[
        pltpu.VMEM((_BPW + _L,), jnp.int32),
        pltpu.VMEM((_R, _D, _TL), jnp.float32),
        pltpu.VMEM((_D, _BPW), jnp.float32),
        pltpu.SemaphoreType.DMA,
    ],
    compiler_params=pltpu.CompilerParams(needs_layout_passes=False),
)
def _gather_cols(vt_hbm, idx_hbm, out_hbm, idx_s, slabs, cols_v, sem):
    wid = lax.axis_index("s") * _NUM_CORES + lax.axis_index("c")
    base = pl.multiple_of(wid * _BPW, _BPW)
    pltpu.sync_copy(idx_hbm.at[pl.ds(base, _BPW)], idx_s.at[pl.ds(0, _BPW)])

    def idx_at(k):
        return idx_s[pl.ds(k, _L)][0]

    def fetch(k, r):
        start = pl.multiple_of(
            lax.shift_right_logical(idx_at(k), 7) * _TL, _TL
        )
        pltpu.make_async_copy(
            vt_hbm.at[:, pl.ds(start, _TL)],
            slabs.at[r],
            sem,
        ).start()

    for r in range(_R):
        fetch(r, r)

    j_lo = lax.iota(jnp.int32, _L)
    j_hi = j_lo + _L

    def outer(it, _):
        k0 = it * _R
        for r in range(_R):
            k = k0 + r
            pltpu.make_async_copy(
                vt_hbm.at[:, pl.ds(0, _TL)], slabs.at[r], sem
            ).wait()
            lane = jnp.full((_L,), idx_at(k) & (_TL - 1), jnp.int32)
            r_vec = jnp.full((_L,), r, jnp.int32)
            k_vec = jnp.full((_L,), k, jnp.int32)
            v0 = plsc.load_gather(slabs, [r_vec, j_lo, lane])
            v1 = plsc.load_gather(slabs, [r_vec, j_hi, lane])
            plsc.store_scatter(cols_v, [j_lo, k_vec], v0)
            plsc.store_scatter(cols_v, [j_hi, k_vec], v1)

            @pl.when(k + _R < _BPW)
            def _():
                fetch(k + _R, r)

        return 0

    lax.fori_loop(0, _BPW // _R, outer, 0)
    pltpu.sync_copy(cols_v, out_hbm.at[:, pl.ds(base, _BPW)])


def kernel(indices, values):
    out_t = _gather_cols(values.T, indices.astype(jnp.int32))
    return out_t.T
